# Initial kernel scaffold; baseline (speedup 1.0000x reference)
import jax
import jax.numpy as jnp
from jax.experimental import pallas as pl


def kernel(x, base_bias, cm0, w0, b0, cm1, w1, b1, cm2, w2):
    raise NotImplementedError("write your pallas kernel here")


# speedup vs baseline: 110.3334x; 110.3334x over previous
import math

import jax
import jax.numpy as jnp
from jax.experimental import pallas as pl

_P = 0.001
_CLIP_LO = math.log(_P / (1.0 - _P))
_CLIP_HI = -_CLIP_LO
_BB = 256


def _bits_to_idx(d3):
    return ((d3[0] > 0).astype(jnp.int32)
            + 2 * (d3[1] > 0).astype(jnp.int32)
            + 4 * (d3[2] > 0).astype(jnp.int32)
            + 8 * (d3[3] > 0).astype(jnp.int32))


def _select16(a3, idx):
    j = jax.lax.broadcasted_iota(jnp.int32, a3.shape, 0)
    return jnp.sum(jnp.where(j == idx[None], a3, 0.0), axis=0)


def _gln_body(xT_ref, bb_ref, cm0_ref, w0_ref, cm1_ref, w1m_ref, w1b_ref,
              b0_ref, cm2_ref, w2m_ref, w2b_ref, b1_ref, out_ref):
    xT = xT_ref[...]
    base = jnp.clip(xT, _P, 1.0 - _P)
    l0 = jnp.log(base / (1.0 - base))
    row = jax.lax.broadcasted_iota(jnp.int32, l0.shape, 0)
    l0 = jnp.where(row == 0, jnp.broadcast_to(bb_ref[...], l0.shape), l0)

    d0 = jnp.dot(cm0_ref[...], xT,
                 preferred_element_type=jnp.float32).reshape(4, 128, -1)
    a0 = jnp.dot(w0_ref[...], l0,
                 preferred_element_type=jnp.float32).reshape(16, 128, -1)
    h0 = jnp.clip(_select16(a0, _bits_to_idx(d0)), _CLIP_LO, _CLIP_HI)

    d1 = jnp.dot(cm1_ref[...], xT,
                 preferred_element_type=jnp.float32).reshape(4, 64, -1)
    a1 = jnp.dot(w1m_ref[...], h0, preferred_element_type=jnp.float32)
    a1 = (a1 + w1b_ref[...] * b0_ref[...]).reshape(16, 64, -1)
    h1 = jnp.clip(_select16(a1, _bits_to_idx(d1)), _CLIP_LO, _CLIP_HI)

    d2 = jnp.dot(cm2_ref[...], xT,
                 preferred_element_type=jnp.float32).reshape(4, 8, -1)
    a2 = jnp.dot(w2m_ref[...], h1, preferred_element_type=jnp.float32)
    a2 = (a2 + w2b_ref[...] * b1_ref[...]).reshape(16, 8, -1)
    out_ref[...] = jnp.clip(_select16(a2, _bits_to_idx(d2)), _CLIP_LO, _CLIP_HI)


def _prep_cm(cm, sp):
    m = jnp.transpose(cm[0], (1, 0, 2))
    m = jnp.pad(m, ((0, 0), (0, sp - m.shape[1]), (0, 0)))
    return m.reshape(4 * sp, m.shape[2])


def _prep_w(w, sp, drop_bias_col):
    wm = w[0]
    wb = None
    if drop_bias_col:
        wb = wm[:, :, 0]
        wm = jnp.pad(wm[:, :, 1:], ((0, 0), (0, 0), (0, 1)))
    t = jnp.transpose(wm, (1, 0, 2))
    t = jnp.pad(t, ((0, 0), (0, sp - t.shape[1]), (0, 0)))
    wmp = t.reshape(16 * sp, t.shape[2])
    if drop_bias_col:
        tb = jnp.pad(wb.T, ((0, 0), (0, sp - wb.shape[0])))
        return wmp, tb.reshape(16 * sp, 1)
    return wmp, None


def kernel(x, base_bias, cm0, w0, b0, cm1, w1, b1, cm2, w2):
    B = x.shape[0]
    xT = x.T
    bb = jnp.asarray(base_bias, jnp.float32).reshape(1, 1)
    b0s = b0.reshape(1, 1)
    b1s = b1.reshape(1, 1)

    cm0p = _prep_cm(cm0, 128)
    w0p, _ = _prep_w(w0, 128, False)
    cm1p = _prep_cm(cm1, 64)
    w1m, w1b = _prep_w(w1, 64, True)
    cm2p = _prep_cm(cm2, 8)
    w2m, w2b = _prep_w(w2, 8, True)

    def fixed(a):
        return pl.BlockSpec(a.shape, lambda i: (0, 0))

    out = pl.pallas_call(
        _gln_body,
        grid=(B // _BB,),
        in_specs=[
            pl.BlockSpec((x.shape[1], _BB), lambda i: (0, i)),
            fixed(bb), fixed(cm0p), fixed(w0p), fixed(cm1p),
            fixed(w1m), fixed(w1b), fixed(b0s),
            fixed(cm2p), fixed(w2m), fixed(w2b), fixed(b1s),
        ],
        out_specs=pl.BlockSpec((8, _BB), lambda i: (0, i)),
        out_shape=jax.ShapeDtypeStruct((8, B), jnp.float32),
    )(xT, bb, cm0p, w0p, cm1p, w1m, w1b, b0s, cm2p, w2m, w2b, b1s)
    return out[0].reshape(B, 1, 1)


# speedup vs baseline: 169.2613x; 1.5341x over previous
import math

import jax
import jax.numpy as jnp
from jax.experimental import pallas as pl

_P = 0.001
_CLIP_LO = math.log(_P / (1.0 - _P))
_CLIP_HI = -_CLIP_LO
_BB = 1024

_CM_END = 800
_W0_END = _CM_END + 16 * 128
_W1_END = _W0_END + 16 * 64
_W2_END = _W1_END + 16 * 8


def _tree_select(a3, d3):
    m0, m1, m2, m3 = (d3[0] > 0, d3[1] > 0, d3[2] > 0, d3[3] > 0)
    t = jnp.where(m3[None], a3[8:16], a3[0:8])
    t = jnp.where(m2[None], t[4:8], t[0:4])
    t = jnp.where(m1[None], t[2:4], t[0:2])
    t = jnp.where(m0, t[1], t[0])
    return jnp.clip(t, _CLIP_LO, _CLIP_HI)


def _set_row(h, r0, bias):
    row = jax.lax.broadcasted_iota(jnp.int32, h.shape, 0)
    return jnp.where(row == r0, jnp.broadcast_to(bias, h.shape), h)


def _dot_nk(a, b):
    return jax.lax.dot_general(a, b, (((1,), (1,)), ((), ())),
                               preferred_element_type=jnp.float32)


def _gln_body(x_ref, bb_ref, buf_ref, b0_ref, b1_ref, out_ref):
    x = x_ref[...]
    base = jnp.clip(x, _P, 1.0 - _P)
    l0 = jnp.log(base / (1.0 - base))
    col = jax.lax.broadcasted_iota(jnp.int32, l0.shape, 1)
    l0 = jnp.where(col == 0, jnp.broadcast_to(bb_ref[...], l0.shape), l0)

    d = _dot_nk(buf_ref[0:_CM_END], x)
    d0 = d[0:512].reshape(4, 128, -1)
    d1 = d[512:768].reshape(4, 64, -1)
    d2 = d[768:800].reshape(4, 8, -1)

    a0 = _dot_nk(buf_ref[_CM_END:_W0_END], l0).reshape(16, 128, -1)
    h0 = _tree_select(a0, d0)

    l1 = _set_row(h0, 127, b0_ref[...])
    a1 = jnp.dot(buf_ref[_W0_END:_W1_END, 0:128], l1,
                 preferred_element_type=jnp.float32).reshape(16, 64, -1)
    h1 = _tree_select(a1, d1)

    l2 = _set_row(h1, 63, b1_ref[...])
    a2 = jnp.dot(buf_ref[_W1_END:_W2_END, 0:64], l2,
                 preferred_element_type=jnp.float32).reshape(16, 8, -1)
    out_ref[...] = _tree_select(a2, d2)


def _prep_mj(t, sp, width):
    t = jnp.transpose(t, (1, 0, 2))
    t = jnp.pad(t, ((0, 0), (0, sp - t.shape[1]), (0, width - t.shape[2])))
    return t.reshape(t.shape[0] * sp, width)


def kernel(x, base_bias, cm0, w0, b0, cm1, w1, b1, cm2, w2):
    B = x.shape[0]
    bb = jnp.asarray(base_bias, jnp.float32).reshape(1, 1)
    b0s = b0.reshape(1, 1)
    b1s = b1.reshape(1, 1)

    buf = jnp.concatenate([
        _prep_mj(cm0[0], 128, 256), _prep_mj(cm1[0], 64, 256),
        _prep_mj(cm2[0], 8, 256), _prep_mj(w0[0], 128, 256),
        _prep_mj(jnp.roll(w1[0], -1, axis=2), 64, 256),
        _prep_mj(jnp.roll(w2[0], -1, axis=2), 8, 256),
    ], axis=0)

    def fixed(a):
        return pl.BlockSpec(a.shape, lambda i: (0, 0))

    out = pl.pallas_call(
        _gln_body,
        grid=(B // _BB,),
        in_specs=[
            pl.BlockSpec((_BB, x.shape[1]), lambda i: (i, 0)),
            fixed(bb), fixed(buf), fixed(b0s), fixed(b1s),
        ],
        out_specs=pl.BlockSpec((8, _BB), lambda i: (0, i)),
        out_shape=jax.ShapeDtypeStruct((8, B), jnp.float32),
    )(x, bb, buf, b0s, b1s)
    return out[0].reshape(B, 1, 1)


# speedup vs baseline: 193.8047x; 1.1450x over previous
import math

import jax
import jax.numpy as jnp
from jax.experimental import pallas as pl

_P = 0.001
_CLIP_LO = math.log(_P / (1.0 - _P))
_CLIP_HI = -_CLIP_LO
_BB = 1024

_CM_END = 800
_W0_END = _CM_END + 16 * 128
_W1_END = _W0_END + 16 * 64
_W2_END = _W1_END + 16 * 8


def _tree_select(a3, d3):
    m0, m1, m2, m3 = (d3[0] > 0, d3[1] > 0, d3[2] > 0, d3[3] > 0)
    t = jnp.where(m3[None], a3[8:16], a3[0:8])
    t = jnp.where(m2[None], t[4:8], t[0:4])
    t = jnp.where(m1[None], t[2:4], t[0:2])
    t = jnp.where(m0, t[1], t[0])
    return jnp.clip(t, _CLIP_LO, _CLIP_HI)


def _set_row(h, r0, bias):
    row = jax.lax.broadcasted_iota(jnp.int32, h.shape, 0)
    return jnp.where(row == r0, jnp.broadcast_to(bias, h.shape), h)


def _dot_nk(a, b):
    return jax.lax.dot_general(a, b, (((1,), (1,)), ((), ())),
                               preferred_element_type=jnp.float32)


def _gln_body(x_ref, bb_ref, buf_ref, b0_ref, b1_ref, out_ref):
    x = x_ref[...]
    base = jnp.clip(x, _P, 1.0 - _P)
    l0 = jnp.log(base / (1.0 - base))
    col = jax.lax.broadcasted_iota(jnp.int32, l0.shape, 1)
    l0 = jnp.where(col == 0, jnp.broadcast_to(bb_ref[...], l0.shape), l0)

    xb = x.astype(jnp.bfloat16)
    d = _dot_nk(buf_ref[0:_CM_END], xb)
    d0 = d[0:512].reshape(4, 128, -1)
    d1 = d[512:768].reshape(4, 64, -1)
    d2 = d[768:800].reshape(4, 8, -1)

    a0 = _dot_nk(buf_ref[_CM_END:_W0_END],
                 l0.astype(jnp.bfloat16)).reshape(16, 128, -1)
    h0 = _tree_select(a0, d0)

    l1 = _set_row(h0, 127, b0_ref[...])
    a1 = jnp.dot(buf_ref[_W0_END:_W1_END, 0:128], l1.astype(jnp.bfloat16),
                 preferred_element_type=jnp.float32).reshape(16, 64, -1)
    h1 = _tree_select(a1, d1)

    l2 = _set_row(h1, 63, b1_ref[...])
    a2 = jnp.dot(buf_ref[_W1_END:_W2_END, 0:64], l2.astype(jnp.bfloat16),
                 preferred_element_type=jnp.float32).reshape(16, 8, -1)
    o = _tree_select(a2, d2)[0:1]
    out_ref[...] = jnp.transpose(o)


def _prep_mj(t, sp, width):
    t = jnp.transpose(t, (1, 0, 2))
    t = jnp.pad(t, ((0, 0), (0, sp - t.shape[1]), (0, width - t.shape[2])))
    return t.reshape(t.shape[0] * sp, width).astype(jnp.bfloat16)


def kernel(x, base_bias, cm0, w0, b0, cm1, w1, b1, cm2, w2):
    B = x.shape[0]
    bb = jnp.asarray(base_bias, jnp.float32).reshape(1, 1)
    b0s = b0.reshape(1, 1)
    b1s = b1.reshape(1, 1)

    buf = jnp.concatenate([
        _prep_mj(cm0[0], 128, 256), _prep_mj(cm1[0], 64, 256),
        _prep_mj(cm2[0], 8, 256), _prep_mj(w0[0], 128, 256),
        _prep_mj(jnp.roll(w1[0], -1, axis=2), 64, 256),
        _prep_mj(jnp.roll(w2[0], -1, axis=2), 8, 256),
    ], axis=0)

    def fixed(a):
        return pl.BlockSpec(a.shape, lambda i: (0, 0))

    out = pl.pallas_call(
        _gln_body,
        grid=(B // _BB,),
        in_specs=[
            pl.BlockSpec((_BB, x.shape[1]), lambda i: (i, 0)),
            fixed(bb), fixed(buf), fixed(b0s), fixed(b1s),
        ],
        out_specs=pl.BlockSpec((_BB, 1), lambda i: (i, 0)),
        out_shape=jax.ShapeDtypeStruct((B, 1), jnp.float32),
    )(x, bb, buf, b0s, b1s)
    return out[:, :, None]
